# baseline (speedup 1.0000x reference)
import functools

import jax
import jax.numpy as jnp
from jax import lax
from jax.experimental import pallas as pl
from jax.experimental.pallas import tpu as pltpu
from jax.experimental.pallas import tpu_sc as plsc

N_NODES = 50000
NPAD = 50176
BN = 512
NBLK = NPAD // BN
NSEG = 16
NC, NS = 2, 16
IB = 128
NB = 4
SCH = 40
NSUP = 10
TOT_B = SCH * NSUP
E_PER_TILE = TOT_B * IB
EP = E_PER_TILE * NS
RPT = NPAD // NS
CW = 32


def _dense_layer(h_in, nbr_chunks, w0t, b0, w1t, b1):
    n_nbr = len(nbr_chunks)
    din, dout = w0t.shape
    cout = dout // CW

    def body(*refs):
        h_ref = refs[0]
        nbrs = refs[1:1 + n_nbr]
        w0t_r, b0_r, w1t_r, b1_r = refs[1 + n_nbr:5 + n_nbr]
        outs = refs[5 + n_nbr:]
        h = h_ref[...]
        if n_nbr:
            nbr = jnp.concatenate([r[...] for r in nbrs], axis=1)
            h = jnp.maximum(h + nbr, 0.0)
        vw0 = jnp.dot(h, w0t_r[...], preferred_element_type=jnp.float32) + b0_r[...]
        vw1 = jnp.dot(h, w1t_r[...], preferred_element_type=jnp.float32) + b1_r[...]
        outs[0][...] = vw0
        for c in range(cout):
            outs[1 + c][...] = vw1[:, c * CW:(c + 1) * CW]

    row_spec = lambda w: pl.BlockSpec((BN, w), lambda i: (i, 0))
    full_spec = lambda a: pl.BlockSpec(a.shape, lambda i: (0, 0))
    in_specs = ([row_spec(din)] + [row_spec(CW)] * n_nbr
                + [full_spec(w0t), full_spec(b0), full_spec(w1t), full_spec(b1)])
    out_specs = [row_spec(dout)] + [row_spec(CW)] * cout
    out_shape = ([jax.ShapeDtypeStruct((NPAD, dout), jnp.float32)]
                 + [jax.ShapeDtypeStruct((NPAD, CW), jnp.float32)] * cout)
    res = pl.pallas_call(
        body, grid=(NBLK,), in_specs=in_specs, out_specs=out_specs,
        out_shape=out_shape,
    )(h_in, *nbr_chunks, w0t, b0, w1t, b1)
    return res[0], list(res[1:])


def _make_scatter(n_chunks):
    n_pass = n_chunks // NC
    mesh = plsc.VectorSubcoreMesh(core_axis_name="c", subcore_axis_name="s")
    out_type = [jax.ShapeDtypeStruct((NPAD, CW), jnp.float32)] * n_chunks
    scratch = [
        pltpu.VMEM((SCH, IB), jnp.int32),
        pltpu.VMEM((SCH, IB), jnp.int32),
        pltpu.VMEM((NB * IB, CW), jnp.float32),
        pltpu.VMEM_SHARED((NPAD, CW), jnp.float32),
        pltpu.SemaphoreType.DMA,
        pltpu.SemaphoreType.DMA,
    ]

    @functools.partial(pl.kernel, out_type=out_type, mesh=mesh,
                       scratch_types=scratch,
                       compiler_params=pltpu.CompilerParams(
                           use_tc_tiling_on_sc=False))
    def sc_scatter(*refs):
        tables = refs[:n_chunks]
        dst_hbm, src_hbm, zeros_hbm = refs[n_chunks:n_chunks + 3]
        outs = refs[n_chunks + 3:2 * n_chunks + 3]
        dstb, srcb, rows, acc, gsem, ssem = refs[2 * n_chunks + 3:]
        cid = lax.axis_index("c")
        sid = lax.axis_index("s")
        my_rows = pl.ds(sid * RPT, RPT)
        for p in range(n_pass):
            for cc in range(NC):
                c = p * NC + cc

                @pl.when(cid == cc)
                def _(c=c):
                    pltpu.sync_copy(zeros_hbm, acc.at[my_rows])
                    plsc.subcore_barrier()

                    def sup_body(si, _):
                        row0 = sid * TOT_B + si * SCH
                        pltpu.sync_copy(dst_hbm.at[pl.ds(row0, SCH)], dstb)
                        pltpu.sync_copy(src_hbm.at[pl.ds(row0, SCH)], srcb)
                        pltpu.async_copy(tables[c].at[srcb.at[0]],
                                         rows.at[pl.ds(0, IB)], gsem)
                        pltpu.async_copy(tables[c].at[srcb.at[1]],
                                         rows.at[pl.ds(IB, IB)], gsem)

                        def b_body(b, _):
                            @pl.when(b >= 2)
                            def _():
                                pltpu.make_async_copy(
                                    rows.at[pl.ds(0, IB)],
                                    acc.at[pl.ds(0, IB)], ssem).wait()

                            @pl.when(b < SCH - 2)
                            def _():
                                slot = ((b + 2) % NB) * IB
                                pltpu.async_copy(
                                    tables[c].at[srcb.at[b + 2]],
                                    rows.at[pl.ds(slot, IB)], gsem)

                            pltpu.make_async_copy(
                                zeros_hbm.at[pl.ds(0, IB)],
                                rows.at[pl.ds(0, IB)], gsem).wait()
                            pltpu.async_copy(
                                rows.at[pl.ds((b % NB) * IB, IB)],
                                acc.at[dstb.at[b]], ssem, add=True)
                            return 0

                        lax.fori_loop(0, SCH, b_body, 0)
                        for _ in range(2):
                            pltpu.make_async_copy(
                                rows.at[pl.ds(0, IB)],
                                acc.at[pl.ds(0, IB)], ssem).wait()
                        return 0

                    lax.fori_loop(0, NSUP, sup_body, 0)
                    plsc.subcore_barrier()
                    pltpu.sync_copy(acc.at[my_rows], outs[c].at[my_rows])
                    plsc.subcore_barrier()

    return sc_scatter


_make_scatter = functools.lru_cache(maxsize=None)(_make_scatter)


def _scatter2(*args):
    return _make_scatter(2)(*args)


def _scatter4(*args):
    return _make_scatter(4)(*args)


def _pool_head(vw0, nbr_chunks, m_col, h1wt, h1b, h2wt, h2b):
    n_nbr = len(nbr_chunks)
    dim = vw0.shape[1]

    def body(*refs):
        vw0_r = refs[0]
        nbrs = refs[1:1 + n_nbr]
        m_r, h1wt_r, h1b_r, h2wt_r, h2b_r = refs[1 + n_nbr:6 + n_nbr]
        out_r = refs[6 + n_nbr]
        sums, counts = refs[7 + n_nbr:]
        i = pl.program_id(0)

        @pl.when(i == 0)
        def _():
            sums[...] = jnp.zeros_like(sums)
            counts[...] = jnp.zeros_like(counts)
            out_r[...] = jnp.zeros_like(out_r)

        nbr = jnp.concatenate([r[...] for r in nbrs], axis=1)
        h = jnp.maximum(vw0_r[...] + nbr, 0.0)
        seg_ids = lax.broadcasted_iota(jnp.int32, (1, NSEG), 1).astype(jnp.float32)
        seg = (m_r[...] == seg_ids)
        seg = seg.astype(jnp.float32)
        dn = (((0,), (0,)), ((), ()))
        sums[...] += lax.dot_general(seg, h, dn,
                                     preferred_element_type=jnp.float32)
        counts[...] += lax.dot_general(
            seg, jnp.ones((BN, dim), jnp.float32), dn,
            preferred_element_type=jnp.float32)

        @pl.when(i == NBLK - 1)
        def _():
            gfeat = sums[...] / jnp.maximum(counts[...], 1e-6)
            t = jnp.dot(gfeat, h1wt_r[...], preferred_element_type=jnp.float32)
            t = jnp.maximum(t + h1b_r[...], 0.0)
            out_r[...] = (jnp.dot(t, h2wt_r[...],
                                  preferred_element_type=jnp.float32)
                          + h2b_r[...])

    row_spec = lambda w: pl.BlockSpec((BN, w), lambda i: (i, 0))
    full_spec = lambda a: pl.BlockSpec(a.shape, lambda i: (0, 0))
    in_specs = ([row_spec(dim)] + [row_spec(CW)] * n_nbr
                + [row_spec(1), full_spec(h1wt), full_spec(h1b),
                   full_spec(h2wt), full_spec(h2b)])
    return pl.pallas_call(
        body, grid=(NBLK,), in_specs=in_specs,
        out_specs=pl.BlockSpec((NSEG, 128), lambda i: (0, 0)),
        out_shape=jax.ShapeDtypeStruct((NSEG, 128), jnp.float32),
        scratch_shapes=[pltpu.VMEM((NSEG, dim), jnp.float32),
                        pltpu.VMEM((NSEG, dim), jnp.float32)],
    )(vw0, *nbr_chunks, m_col, h1wt, h1b, h2wt, h2b)


def kernel(x, edges, m_idx,
           g1_w0, g1_b0, g1_w1, g1_b1,
           g2_w0, g2_b0, g2_w1, g2_b1,
           g3_w0, g3_b0, g3_w1, g3_b1,
           h1_w, h1_b, h2_w, h2_b):
    f32 = jnp.float32
    n, e = x.shape[0], edges.shape[0]

    xpad = jnp.zeros((NPAD, 128), f32).at[:n, :3].set(x)
    dst = jnp.concatenate([edges[:, 0], edges[:, 1]])
    src = jnp.concatenate([edges[:, 1], edges[:, 0]])
    npad_e = EP - 2 * e
    dst = jnp.concatenate([dst, jnp.full((npad_e,), N_NODES, jnp.int32)])
    src = jnp.concatenate([src, jnp.zeros((npad_e,), jnp.int32)])
    dst2 = dst.reshape(EP // IB, IB)
    src2 = src.reshape(EP // IB, IB)
    zeros_sc = jnp.zeros((RPT, CW), f32)
    m_col = jnp.full((NPAD, 1), float(NSEG), f32).at[:n, 0].set(
        m_idx.astype(f32))

    w0t1 = jnp.zeros((128, 64), f32).at[:3].set(g1_w0.T)
    w1t1 = jnp.zeros((128, 64), f32).at[:3].set(g1_w1.T)
    b01, b11 = g1_b0[None, :], g1_b1[None, :]
    w0t2, w1t2 = g2_w0.T, g2_w1.T
    b02, b12 = g2_b0[None, :], g2_b1[None, :]
    w0t3, w1t3 = g3_w0.T, g3_w1.T
    b03, b13 = g3_b0[None, :], g3_b1[None, :]
    h1wt = h1_w.T
    h1b = h1_b[None, :]
    h2wt = jnp.zeros((256, 128), f32).at[:, :h2_w.shape[0]].set(h2_w.T)
    h2b = jnp.zeros((1, 128), f32).at[0, :h2_b.shape[0]].set(h2_b)

    vw0_1, c1 = _dense_layer(xpad, [], w0t1, b01, w1t1, b11)
    nbr1 = _scatter2(*c1, dst2, src2, zeros_sc)
    vw0_2, c2 = _dense_layer(vw0_1, list(nbr1), w0t2, b02, w1t2, b12)
    nbr2 = _scatter4(*c2, dst2, src2, zeros_sc)
    vw0_3, c3 = _dense_layer(vw0_2, list(nbr2), w0t3, b03, w1t3, b13)
    nbr3 = _scatter4(*c3, dst2, src2, zeros_sc)
    out = _pool_head(vw0_3, list(nbr3), m_col, h1wt, h1b, h2wt, h2b)
    return out[:, :h2_w.shape[0]]


# speedup vs baseline: 1.0491x; 1.0491x over previous
import functools

import jax
import jax.numpy as jnp
from jax import lax
from jax.experimental import pallas as pl
from jax.experimental.pallas import tpu as pltpu
from jax.experimental.pallas import tpu_sc as plsc

N_NODES = 50000
NPAD = 50176
BN = 1024
NBLK = NPAD // BN
NSEG = 16
NC, NS = 2, 16
IB = 128
NB = 6
SCH = 20
NSUP = 20
TOT_B = SCH * NSUP
E_PER_TILE = TOT_B * IB
EP = E_PER_TILE * NS
RPT = NPAD // NS
CW = 32


def _dense_layer(h_in, nbr_chunks, w0t, b0, w1t, b1):
    n_nbr = len(nbr_chunks)
    din, dout = w0t.shape
    cout = dout // CW

    def body(*refs):
        h_ref = refs[0]
        nbrs = refs[1:1 + n_nbr]
        w0t_r, b0_r, w1t_r, b1_r = refs[1 + n_nbr:5 + n_nbr]
        outs = refs[5 + n_nbr:]
        h = h_ref[...]
        if n_nbr:
            nbr = jnp.concatenate([r[...] for r in nbrs], axis=1)
            h = jnp.maximum(h + nbr, 0.0)
        vw0 = jnp.dot(h, w0t_r[...], preferred_element_type=jnp.float32) + b0_r[...]
        vw1 = jnp.dot(h, w1t_r[...], preferred_element_type=jnp.float32) + b1_r[...]
        outs[0][...] = vw0
        for c in range(cout):
            outs[1 + c][...] = vw1[:, c * CW:(c + 1) * CW]

    row_spec = lambda w: pl.BlockSpec((BN, w), lambda i: (i, 0))
    full_spec = lambda a: pl.BlockSpec(a.shape, lambda i: (0, 0))
    in_specs = ([row_spec(din)] + [row_spec(CW)] * n_nbr
                + [full_spec(w0t), full_spec(b0), full_spec(w1t), full_spec(b1)])
    out_specs = [row_spec(dout)] + [row_spec(CW)] * cout
    out_shape = ([jax.ShapeDtypeStruct((NPAD, dout), jnp.float32)]
                 + [jax.ShapeDtypeStruct((NPAD, CW), jnp.float32)] * cout)
    res = pl.pallas_call(
        body, grid=(NBLK,), in_specs=in_specs, out_specs=out_specs,
        out_shape=out_shape,
    )(h_in, *nbr_chunks, w0t, b0, w1t, b1)
    return res[0], list(res[1:])


def _make_scatter(n_chunks):
    n_pass = n_chunks // NC
    mesh = plsc.VectorSubcoreMesh(core_axis_name="c", subcore_axis_name="s")
    out_type = [jax.ShapeDtypeStruct((NPAD, CW), jnp.float32)] * n_chunks
    scratch = [
        pltpu.VMEM((SCH, IB), jnp.int32),
        pltpu.VMEM((SCH, IB), jnp.int32),
        pltpu.VMEM((NB * IB, CW), jnp.float32),
        pltpu.VMEM_SHARED((NPAD, CW), jnp.float32),
        pltpu.SemaphoreType.DMA,
        pltpu.SemaphoreType.DMA,
    ]

    @functools.partial(pl.kernel, out_type=out_type, mesh=mesh,
                       scratch_types=scratch,
                       compiler_params=pltpu.CompilerParams(
                           use_tc_tiling_on_sc=False))
    def sc_scatter(*refs):
        tables = refs[:n_chunks]
        dst_hbm, src_hbm, zeros_hbm = refs[n_chunks:n_chunks + 3]
        outs = refs[n_chunks + 3:2 * n_chunks + 3]
        dstb, srcb, rows, acc, gsem, ssem = refs[2 * n_chunks + 3:]
        cid = lax.axis_index("c")
        sid = lax.axis_index("s")
        my_rows = pl.ds(sid * RPT, RPT)
        for p in range(n_pass):
            for cc in range(NC):
                c = p * NC + cc

                @pl.when(cid == cc)
                def _(c=c):
                    pltpu.sync_copy(zeros_hbm, acc.at[my_rows])
                    plsc.subcore_barrier()

                    def sup_body(si, _):
                        row0 = sid * TOT_B + si * SCH
                        pltpu.sync_copy(dst_hbm.at[pl.ds(row0, SCH)], dstb)
                        pltpu.sync_copy(src_hbm.at[pl.ds(row0, SCH)], srcb)
                        for q in range(4):
                            pltpu.async_copy(tables[c].at[srcb.at[q]],
                                             rows.at[pl.ds(q * IB, IB)], gsem)

                        def b_body(b, _):
                            @pl.when(b >= 2)
                            def _():
                                pltpu.make_async_copy(
                                    rows.at[pl.ds(0, IB)],
                                    acc.at[pl.ds(0, IB)], ssem).wait()

                            @pl.when(b < SCH - 4)
                            def _():
                                slot = ((b + 4) % NB) * IB
                                pltpu.async_copy(
                                    tables[c].at[srcb.at[b + 4]],
                                    rows.at[pl.ds(slot, IB)], gsem)

                            pltpu.make_async_copy(
                                zeros_hbm.at[pl.ds(0, IB)],
                                rows.at[pl.ds(0, IB)], gsem).wait()
                            pltpu.async_copy(
                                rows.at[pl.ds((b % NB) * IB, IB)],
                                acc.at[dstb.at[b]], ssem, add=True)
                            return 0

                        lax.fori_loop(0, SCH, b_body, 0)
                        for _ in range(2):
                            pltpu.make_async_copy(
                                rows.at[pl.ds(0, IB)],
                                acc.at[pl.ds(0, IB)], ssem).wait()
                        return 0

                    lax.fori_loop(0, NSUP, sup_body, 0)
                    plsc.subcore_barrier()
                    pltpu.sync_copy(acc.at[my_rows], outs[c].at[my_rows])
                    plsc.subcore_barrier()

    return sc_scatter


_make_scatter = functools.lru_cache(maxsize=None)(_make_scatter)


def _scatter2(*args):
    return _make_scatter(2)(*args)


def _scatter4(*args):
    return _make_scatter(4)(*args)


def _pool_head(vw0, nbr_chunks, m_col, h1wt, h1b, h2wt, h2b):
    n_nbr = len(nbr_chunks)
    dim = vw0.shape[1]

    def body(*refs):
        vw0_r = refs[0]
        nbrs = refs[1:1 + n_nbr]
        m_r, h1wt_r, h1b_r, h2wt_r, h2b_r = refs[1 + n_nbr:6 + n_nbr]
        out_r = refs[6 + n_nbr]
        sums, counts = refs[7 + n_nbr:]
        i = pl.program_id(0)

        @pl.when(i == 0)
        def _():
            sums[...] = jnp.zeros_like(sums)
            counts[...] = jnp.zeros_like(counts)
            out_r[...] = jnp.zeros_like(out_r)

        nbr = jnp.concatenate([r[...] for r in nbrs], axis=1)
        h = jnp.maximum(vw0_r[...] + nbr, 0.0)
        seg_ids = lax.broadcasted_iota(jnp.int32, (1, NSEG), 1).astype(jnp.float32)
        seg = (m_r[...] == seg_ids)
        seg = seg.astype(jnp.float32)
        dn = (((0,), (0,)), ((), ()))
        sums[...] += lax.dot_general(seg, h, dn,
                                     preferred_element_type=jnp.float32)
        counts[...] += lax.dot_general(
            seg, jnp.ones((BN, dim), jnp.float32), dn,
            preferred_element_type=jnp.float32)

        @pl.when(i == NBLK - 1)
        def _():
            gfeat = sums[...] / jnp.maximum(counts[...], 1e-6)
            t = jnp.dot(gfeat, h1wt_r[...], preferred_element_type=jnp.float32)
            t = jnp.maximum(t + h1b_r[...], 0.0)
            out_r[...] = (jnp.dot(t, h2wt_r[...],
                                  preferred_element_type=jnp.float32)
                          + h2b_r[...])

    row_spec = lambda w: pl.BlockSpec((BN, w), lambda i: (i, 0))
    full_spec = lambda a: pl.BlockSpec(a.shape, lambda i: (0, 0))
    in_specs = ([row_spec(dim)] + [row_spec(CW)] * n_nbr
                + [row_spec(1), full_spec(h1wt), full_spec(h1b),
                   full_spec(h2wt), full_spec(h2b)])
    return pl.pallas_call(
        body, grid=(NBLK,), in_specs=in_specs,
        out_specs=pl.BlockSpec((NSEG, 128), lambda i: (0, 0)),
        out_shape=jax.ShapeDtypeStruct((NSEG, 128), jnp.float32),
        scratch_shapes=[pltpu.VMEM((NSEG, dim), jnp.float32),
                        pltpu.VMEM((NSEG, dim), jnp.float32)],
    )(vw0, *nbr_chunks, m_col, h1wt, h1b, h2wt, h2b)


def kernel(x, edges, m_idx,
           g1_w0, g1_b0, g1_w1, g1_b1,
           g2_w0, g2_b0, g2_w1, g2_b1,
           g3_w0, g3_b0, g3_w1, g3_b1,
           h1_w, h1_b, h2_w, h2_b):
    f32 = jnp.float32
    n, e = x.shape[0], edges.shape[0]

    xpad = jnp.zeros((NPAD, 128), f32).at[:n, :3].set(x)
    dst = jnp.concatenate([edges[:, 0], edges[:, 1]])
    src = jnp.concatenate([edges[:, 1], edges[:, 0]])
    npad_e = EP - 2 * e
    dst = jnp.concatenate([dst, jnp.full((npad_e,), N_NODES, jnp.int32)])
    src = jnp.concatenate([src, jnp.zeros((npad_e,), jnp.int32)])
    dst2 = dst.reshape(EP // IB, IB)
    src2 = src.reshape(EP // IB, IB)
    zeros_sc = jnp.zeros((RPT, CW), f32)
    m_col = jnp.full((NPAD, 1), float(NSEG), f32).at[:n, 0].set(
        m_idx.astype(f32))

    w0t1 = jnp.zeros((128, 64), f32).at[:3].set(g1_w0.T)
    w1t1 = jnp.zeros((128, 64), f32).at[:3].set(g1_w1.T)
    b01, b11 = g1_b0[None, :], g1_b1[None, :]
    w0t2, w1t2 = g2_w0.T, g2_w1.T
    b02, b12 = g2_b0[None, :], g2_b1[None, :]
    w0t3, w1t3 = g3_w0.T, g3_w1.T
    b03, b13 = g3_b0[None, :], g3_b1[None, :]
    h1wt = h1_w.T
    h1b = h1_b[None, :]
    h2wt = jnp.zeros((256, 128), f32).at[:, :h2_w.shape[0]].set(h2_w.T)
    h2b = jnp.zeros((1, 128), f32).at[0, :h2_b.shape[0]].set(h2_b)

    vw0_1, c1 = _dense_layer(xpad, [], w0t1, b01, w1t1, b11)
    nbr1 = _scatter2(*c1, dst2, src2, zeros_sc)
    vw0_2, c2 = _dense_layer(vw0_1, list(nbr1), w0t2, b02, w1t2, b12)
    nbr2 = _scatter4(*c2, dst2, src2, zeros_sc)
    vw0_3, c3 = _dense_layer(vw0_2, list(nbr2), w0t3, b03, w1t3, b13)
    nbr3 = _scatter4(*c3, dst2, src2, zeros_sc)
    out = _pool_head(vw0_3, list(nbr3), m_col, h1wt, h1b, h2wt, h2b)
    return out[:, :h2_w.shape[0]]


# speedup vs baseline: 1.0676x; 1.0177x over previous
import functools

import jax
import jax.numpy as jnp
from jax import lax
from jax.experimental import pallas as pl
from jax.experimental.pallas import tpu as pltpu
from jax.experimental.pallas import tpu_sc as plsc

N_NODES = 50000
NPAD = 50176
BN = 1792
NBLK = NPAD // BN
NSEG = 16
NC, NS = 2, 16
IB = 128
NB = 6
SCH = 20
NSUP = 20
TOT_B = SCH * NSUP
E_PER_TILE = TOT_B * IB
EP = E_PER_TILE * NS
RPT = NPAD // NS
CW = 32


def _dense_layer(h_in, nbr_chunks, w0t, b0, w1t, b1):
    n_nbr = len(nbr_chunks)
    din, dout = w0t.shape
    cout = dout // CW

    def body(*refs):
        h_ref = refs[0]
        nbrs = refs[1:1 + n_nbr]
        w0t_r, b0_r, w1t_r, b1_r = refs[1 + n_nbr:5 + n_nbr]
        outs = refs[5 + n_nbr:]
        h = h_ref[...]
        if n_nbr:
            nbr = jnp.concatenate([r[...] for r in nbrs], axis=1)
            h = jnp.maximum(h + nbr, 0.0)
        vw0 = jnp.dot(h, w0t_r[...], preferred_element_type=jnp.float32) + b0_r[...]
        vw1 = jnp.dot(h, w1t_r[...], preferred_element_type=jnp.float32) + b1_r[...]
        outs[0][...] = vw0
        for c in range(cout):
            outs[1 + c][...] = vw1[:, c * CW:(c + 1) * CW]

    row_spec = lambda w: pl.BlockSpec((BN, w), lambda i: (i, 0))
    full_spec = lambda a: pl.BlockSpec(a.shape, lambda i: (0, 0))
    in_specs = ([row_spec(din)] + [row_spec(CW)] * n_nbr
                + [full_spec(w0t), full_spec(b0), full_spec(w1t), full_spec(b1)])
    out_specs = [row_spec(dout)] + [row_spec(CW)] * cout
    out_shape = ([jax.ShapeDtypeStruct((NPAD, dout), jnp.float32)]
                 + [jax.ShapeDtypeStruct((NPAD, CW), jnp.float32)] * cout)
    res = pl.pallas_call(
        body, grid=(NBLK,), in_specs=in_specs, out_specs=out_specs,
        out_shape=out_shape,
    )(h_in, *nbr_chunks, w0t, b0, w1t, b1)
    return res[0], list(res[1:])


def _make_scatter(n_chunks):
    n_pass = n_chunks // NC
    mesh = plsc.VectorSubcoreMesh(core_axis_name="c", subcore_axis_name="s")
    out_type = [jax.ShapeDtypeStruct((NPAD, CW), jnp.float32)] * n_chunks
    scratch = [
        pltpu.VMEM((SCH, IB), jnp.int32),
        pltpu.VMEM((SCH, IB), jnp.int32),
        pltpu.VMEM((NB * IB, CW), jnp.float32),
        pltpu.VMEM_SHARED((NPAD, CW), jnp.float32),
        pltpu.SemaphoreType.DMA,
        pltpu.SemaphoreType.DMA,
    ]

    @functools.partial(pl.kernel, out_type=out_type, mesh=mesh,
                       scratch_types=scratch,
                       compiler_params=pltpu.CompilerParams(
                           use_tc_tiling_on_sc=False))
    def sc_scatter(*refs):
        tables = refs[:n_chunks]
        dst_hbm, src_hbm, zeros_hbm = refs[n_chunks:n_chunks + 3]
        outs = refs[n_chunks + 3:2 * n_chunks + 3]
        dstb, srcb, rows, acc, gsem, ssem = refs[2 * n_chunks + 3:]
        cid = lax.axis_index("c")
        sid = lax.axis_index("s")
        my_rows = pl.ds(sid * RPT, RPT)
        for p in range(n_pass):
            for cc in range(NC):
                c = p * NC + cc

                @pl.when(cid == cc)
                def _(c=c):
                    pltpu.sync_copy(zeros_hbm, acc.at[my_rows])
                    plsc.subcore_barrier()

                    def sup_body(si, _):
                        row0 = sid * TOT_B + si * SCH
                        pltpu.sync_copy(dst_hbm.at[pl.ds(row0, SCH)], dstb)
                        pltpu.sync_copy(src_hbm.at[pl.ds(row0, SCH)], srcb)
                        for q in range(4):
                            pltpu.async_copy(tables[c].at[srcb.at[q]],
                                             rows.at[pl.ds(q * IB, IB)], gsem)

                        def b_body(b, _):
                            @pl.when(b >= 2)
                            def _():
                                pltpu.make_async_copy(
                                    rows.at[pl.ds(0, IB)],
                                    acc.at[pl.ds(0, IB)], ssem).wait()

                            @pl.when(b < SCH - 4)
                            def _():
                                slot = ((b + 4) % NB) * IB
                                pltpu.async_copy(
                                    tables[c].at[srcb.at[b + 4]],
                                    rows.at[pl.ds(slot, IB)], gsem)

                            pltpu.make_async_copy(
                                zeros_hbm.at[pl.ds(0, IB)],
                                rows.at[pl.ds(0, IB)], gsem).wait()
                            pltpu.async_copy(
                                rows.at[pl.ds((b % NB) * IB, IB)],
                                acc.at[dstb.at[b]], ssem, add=True)
                            return 0

                        lax.fori_loop(0, SCH, b_body, 0)
                        for _ in range(2):
                            pltpu.make_async_copy(
                                rows.at[pl.ds(0, IB)],
                                acc.at[pl.ds(0, IB)], ssem).wait()
                        return 0

                    lax.fori_loop(0, NSUP, sup_body, 0)
                    plsc.subcore_barrier()
                    pltpu.sync_copy(acc.at[my_rows], outs[c].at[my_rows])
                    plsc.subcore_barrier()

    return sc_scatter


_make_scatter = functools.lru_cache(maxsize=None)(_make_scatter)


def _scatter2(*args):
    return _make_scatter(2)(*args)


def _scatter4(*args):
    return _make_scatter(4)(*args)


def _pool_head(vw0, nbr_chunks, m_col, h1wt, h1b, h2wt, h2b):
    n_nbr = len(nbr_chunks)
    dim = vw0.shape[1]

    def body(*refs):
        vw0_r = refs[0]
        nbrs = refs[1:1 + n_nbr]
        m_r, h1wt_r, h1b_r, h2wt_r, h2b_r = refs[1 + n_nbr:6 + n_nbr]
        out_r = refs[6 + n_nbr]
        sums, counts = refs[7 + n_nbr:]
        i = pl.program_id(0)

        @pl.when(i == 0)
        def _():
            sums[...] = jnp.zeros_like(sums)
            counts[...] = jnp.zeros_like(counts)
            out_r[...] = jnp.zeros_like(out_r)

        nbr = jnp.concatenate([r[...] for r in nbrs], axis=1)
        h = jnp.maximum(vw0_r[...] + nbr, 0.0)
        seg_ids = lax.broadcasted_iota(jnp.int32, (1, NSEG), 1).astype(jnp.float32)
        seg = (m_r[...] == seg_ids)
        seg = seg.astype(jnp.float32)
        dn = (((0,), (0,)), ((), ()))
        sums[...] += lax.dot_general(seg, h, dn,
                                     preferred_element_type=jnp.float32)
        counts[...] += lax.dot_general(
            seg, jnp.ones((BN, dim), jnp.float32), dn,
            preferred_element_type=jnp.float32)

        @pl.when(i == NBLK - 1)
        def _():
            gfeat = sums[...] / jnp.maximum(counts[...], 1e-6)
            t = jnp.dot(gfeat, h1wt_r[...], preferred_element_type=jnp.float32)
            t = jnp.maximum(t + h1b_r[...], 0.0)
            out_r[...] = (jnp.dot(t, h2wt_r[...],
                                  preferred_element_type=jnp.float32)
                          + h2b_r[...])

    row_spec = lambda w: pl.BlockSpec((BN, w), lambda i: (i, 0))
    full_spec = lambda a: pl.BlockSpec(a.shape, lambda i: (0, 0))
    in_specs = ([row_spec(dim)] + [row_spec(CW)] * n_nbr
                + [row_spec(1), full_spec(h1wt), full_spec(h1b),
                   full_spec(h2wt), full_spec(h2b)])
    return pl.pallas_call(
        body, grid=(NBLK,), in_specs=in_specs,
        out_specs=pl.BlockSpec((NSEG, 128), lambda i: (0, 0)),
        out_shape=jax.ShapeDtypeStruct((NSEG, 128), jnp.float32),
        scratch_shapes=[pltpu.VMEM((NSEG, dim), jnp.float32),
                        pltpu.VMEM((NSEG, dim), jnp.float32)],
    )(vw0, *nbr_chunks, m_col, h1wt, h1b, h2wt, h2b)


def kernel(x, edges, m_idx,
           g1_w0, g1_b0, g1_w1, g1_b1,
           g2_w0, g2_b0, g2_w1, g2_b1,
           g3_w0, g3_b0, g3_w1, g3_b1,
           h1_w, h1_b, h2_w, h2_b):
    f32 = jnp.float32
    n, e = x.shape[0], edges.shape[0]

    xpad = jnp.zeros((NPAD, 128), f32).at[:n, :3].set(x)
    dst = jnp.concatenate([edges[:, 0], edges[:, 1]])
    src = jnp.concatenate([edges[:, 1], edges[:, 0]])
    npad_e = EP - 2 * e
    dst = jnp.concatenate([dst, jnp.full((npad_e,), N_NODES, jnp.int32)])
    src = jnp.concatenate([src, jnp.zeros((npad_e,), jnp.int32)])
    dst2 = dst.reshape(EP // IB, IB)
    src2 = src.reshape(EP // IB, IB)
    zeros_sc = jnp.zeros((RPT, CW), f32)
    m_col = jnp.full((NPAD, 1), float(NSEG), f32).at[:n, 0].set(
        m_idx.astype(f32))

    w0t1 = jnp.zeros((128, 64), f32).at[:3].set(g1_w0.T)
    w1t1 = jnp.zeros((128, 64), f32).at[:3].set(g1_w1.T)
    b01, b11 = g1_b0[None, :], g1_b1[None, :]
    w0t2, w1t2 = g2_w0.T, g2_w1.T
    b02, b12 = g2_b0[None, :], g2_b1[None, :]
    w0t3, w1t3 = g3_w0.T, g3_w1.T
    b03, b13 = g3_b0[None, :], g3_b1[None, :]
    h1wt = h1_w.T
    h1b = h1_b[None, :]
    h2wt = jnp.zeros((256, 128), f32).at[:, :h2_w.shape[0]].set(h2_w.T)
    h2b = jnp.zeros((1, 128), f32).at[0, :h2_b.shape[0]].set(h2_b)

    vw0_1, c1 = _dense_layer(xpad, [], w0t1, b01, w1t1, b11)
    nbr1 = _scatter2(*c1, dst2, src2, zeros_sc)
    vw0_2, c2 = _dense_layer(vw0_1, list(nbr1), w0t2, b02, w1t2, b12)
    nbr2 = _scatter4(*c2, dst2, src2, zeros_sc)
    vw0_3, c3 = _dense_layer(vw0_2, list(nbr2), w0t3, b03, w1t3, b13)
    nbr3 = _scatter4(*c3, dst2, src2, zeros_sc)
    out = _pool_head(vw0_3, list(nbr3), m_col, h1wt, h1b, h2wt, h2b)
    return out[:, :h2_w.shape[0]]
